# Initial kernel scaffold; baseline (speedup 1.0000x reference)
import jax
import jax.numpy as jnp
from jax import lax
from jax.experimental import pallas as pl
from jax.experimental.pallas import tpu as pltpu
from jax.experimental.pallas import tpu_sc as plsc

_MARGIN = 2.0
_BATCH = 4096
_DIM = 128

_NC = 2
_NS = 16
_NW = _NC * _NS
_BPW = _BATCH // _NW
_LANES = 16


def _sc_partials(ent_hbm, rel_hbm,
                 ph_hbm, pt_hbm, pr_hbm, nh_hbm, nt_hbm, nr_hbm,
                 out_hbm,
                 ph_v, pt_v, pr_v, nh_v, nt_v, nr_v,
                 h_v, t_v, r_v, hn_v, tn_v, rn_v,
                 res_v, sem):
    wid = lax.axis_index("s") * _NC + lax.axis_index("c")
    base = wid * _BPW

    pltpu.sync_copy(ph_hbm.at[pl.ds(base, _BPW)], ph_v)
    pltpu.sync_copy(pt_hbm.at[pl.ds(base, _BPW)], pt_v)
    pltpu.sync_copy(pr_hbm.at[pl.ds(base, _BPW)], pr_v)
    pltpu.sync_copy(nh_hbm.at[pl.ds(base, _BPW)], nh_v)
    pltpu.sync_copy(nt_hbm.at[pl.ds(base, _BPW)], nt_v)
    pltpu.sync_copy(nr_hbm.at[pl.ds(base, _BPW)], nr_v)

    c1 = pltpu.async_copy(ent_hbm.at[ph_v], h_v, sem)
    c2 = pltpu.async_copy(ent_hbm.at[pt_v], t_v, sem)
    c3 = pltpu.async_copy(rel_hbm.at[pr_v], r_v, sem)
    c4 = pltpu.async_copy(ent_hbm.at[nh_v], hn_v, sem)
    c5 = pltpu.async_copy(ent_hbm.at[nt_v], tn_v, sem)
    c6 = pltpu.async_copy(rel_hbm.at[nr_v], rn_v, sem)
    c1.wait(); c2.wait(); c3.wait(); c4.wait(); c5.wait(); c6.wait()

    def body(i, loss):
        acc = jnp.zeros((_LANES,), jnp.float32)
        for d in range(_DIM // _LANES):
            sl = pl.ds(d * _LANES, _LANES)
            acc += jnp.abs(h_v[i, sl] + r_v[i, sl] - t_v[i, sl])
            acc -= jnp.abs(hn_v[i, sl] + rn_v[i, sl] - tn_v[i, sl])
        diff = jnp.sum(acc)
        return loss + jnp.maximum(_MARGIN + diff, 0.0)

    loss = lax.fori_loop(0, _BPW, body, jnp.float32(0.0))

    lane = lax.iota(jnp.int32, _LANES)
    res_v[...] = jnp.where(lane == 0, jnp.full((_LANES,), loss), 0.0)
    pltpu.sync_copy(res_v, out_hbm.at[wid])


@jax.jit
def kernel(entity_vec, relation_vec, pos_h, pos_t, pos_r, neg_h, neg_t, neg_r):
    mesh = plsc.VectorSubcoreMesh(core_axis_name="c", subcore_axis_name="s")
    partials = pl.kernel(
        _sc_partials,
        out_type=jax.ShapeDtypeStruct((_NW, _LANES), jnp.float32),
        mesh=mesh,
        scratch_types=[
            pltpu.VMEM((_BPW,), jnp.int32),
            pltpu.VMEM((_BPW,), jnp.int32),
            pltpu.VMEM((_BPW,), jnp.int32),
            pltpu.VMEM((_BPW,), jnp.int32),
            pltpu.VMEM((_BPW,), jnp.int32),
            pltpu.VMEM((_BPW,), jnp.int32),
            pltpu.VMEM((_BPW, _DIM), jnp.float32),
            pltpu.VMEM((_BPW, _DIM), jnp.float32),
            pltpu.VMEM((_BPW, _DIM), jnp.float32),
            pltpu.VMEM((_BPW, _DIM), jnp.float32),
            pltpu.VMEM((_BPW, _DIM), jnp.float32),
            pltpu.VMEM((_BPW, _DIM), jnp.float32),
            pltpu.VMEM((_LANES,), jnp.float32),
            pltpu.SemaphoreType.DMA,
        ],
    )(entity_vec, relation_vec, pos_h, pos_t, pos_r, neg_h, neg_t, neg_r)

    def _finish(p_ref, o_ref):
        o_ref[0, 0] = jnp.sum(p_ref[...])

    loss = pl.pallas_call(
        _finish,
        out_shape=jax.ShapeDtypeStruct((1, 1), jnp.float32),
        in_specs=[pl.BlockSpec(memory_space=pltpu.VMEM)],
        out_specs=pl.BlockSpec(memory_space=pltpu.SMEM),
    )(partials)
    return loss[0, 0]


# speedup vs baseline: 1.7979x; 1.7979x over previous
import jax
import jax.numpy as jnp
from jax import lax
from jax.experimental import pallas as pl
from jax.experimental.pallas import tpu as pltpu
from jax.experimental.pallas import tpu_sc as plsc

_MARGIN = 2.0
_BATCH = 4096
_DIM = 128

_NC = 2
_NS = 16
_NW = _NC * _NS
_BPW = _BATCH // _NW
_LANES = 16

_TAKE_DNUMS = lax.GatherDimensionNumbers(
    offset_dims=(), collapsed_slice_dims=(0,), start_index_map=(0,))


def _take16(v, idx):
    return lax.gather(v, idx[:, None], _TAKE_DNUMS, slice_sizes=(1,),
                      mode=lax.GatherScatterMode.PROMISE_IN_BOUNDS)


def _sc_partials(ent_hbm, rel_hbm,
                 ph_hbm, pt_hbm, pr_hbm, nh_hbm, nt_hbm, nr_hbm,
                 out_hbm,
                 ph_v, pt_v, pr_v, nh_v, nt_v, nr_v,
                 h_v, t_v, r_v, hn_v, tn_v, rn_v,
                 res_v, sem):
    wid = lax.axis_index("s") * _NC + lax.axis_index("c")
    base = wid * _BPW

    pltpu.sync_copy(ph_hbm.at[pl.ds(base, _BPW)], ph_v)
    pltpu.sync_copy(pt_hbm.at[pl.ds(base, _BPW)], pt_v)
    pltpu.sync_copy(pr_hbm.at[pl.ds(base, _BPW)], pr_v)
    pltpu.sync_copy(nh_hbm.at[pl.ds(base, _BPW)], nh_v)
    pltpu.sync_copy(nt_hbm.at[pl.ds(base, _BPW)], nt_v)
    pltpu.sync_copy(nr_hbm.at[pl.ds(base, _BPW)], nr_v)

    c1 = pltpu.async_copy(ent_hbm.at[ph_v], h_v, sem)
    c2 = pltpu.async_copy(ent_hbm.at[pt_v], t_v, sem)
    c3 = pltpu.async_copy(rel_hbm.at[pr_v], r_v, sem)
    c4 = pltpu.async_copy(ent_hbm.at[nh_v], hn_v, sem)
    c5 = pltpu.async_copy(ent_hbm.at[nt_v], tn_v, sem)
    c6 = pltpu.async_copy(rel_hbm.at[nr_v], rn_v, sem)
    c1.wait(); c2.wait(); c3.wait(); c4.wait(); c5.wait(); c6.wait()

    lane = lax.iota(jnp.int32, _LANES)
    rot = [(lane + s) % _LANES for s in (8, 4, 2, 1)]

    def body(i, loss_vec):
        acc = jnp.zeros((_LANES,), jnp.float32)
        for d in range(_DIM // _LANES):
            sl = pl.ds(d * _LANES, _LANES)
            acc += jnp.abs(h_v[i, sl] + r_v[i, sl] - t_v[i, sl])
            acc -= jnp.abs(hn_v[i, sl] + rn_v[i, sl] - tn_v[i, sl])
        for r in rot:
            acc = acc + _take16(acc, r)
        contrib = jnp.maximum(_MARGIN + acc, 0.0)
        return loss_vec + jnp.where(lane == 0, contrib, 0.0)

    loss_vec = lax.fori_loop(0, _BPW, body, jnp.zeros((_LANES,), jnp.float32))

    res_v[...] = loss_vec
    pltpu.sync_copy(res_v, out_hbm.at[wid])


@jax.jit
def kernel(entity_vec, relation_vec, pos_h, pos_t, pos_r, neg_h, neg_t, neg_r):
    mesh = plsc.VectorSubcoreMesh(core_axis_name="c", subcore_axis_name="s")
    partials = pl.kernel(
        _sc_partials,
        out_type=jax.ShapeDtypeStruct((_NW, _LANES), jnp.float32),
        mesh=mesh,
        scratch_types=[
            pltpu.VMEM((_BPW,), jnp.int32),
            pltpu.VMEM((_BPW,), jnp.int32),
            pltpu.VMEM((_BPW,), jnp.int32),
            pltpu.VMEM((_BPW,), jnp.int32),
            pltpu.VMEM((_BPW,), jnp.int32),
            pltpu.VMEM((_BPW,), jnp.int32),
            pltpu.VMEM((_BPW, _DIM), jnp.float32),
            pltpu.VMEM((_BPW, _DIM), jnp.float32),
            pltpu.VMEM((_BPW, _DIM), jnp.float32),
            pltpu.VMEM((_BPW, _DIM), jnp.float32),
            pltpu.VMEM((_BPW, _DIM), jnp.float32),
            pltpu.VMEM((_BPW, _DIM), jnp.float32),
            pltpu.VMEM((_LANES,), jnp.float32),
            pltpu.SemaphoreType.DMA,
        ],
    )(entity_vec, relation_vec, pos_h, pos_t, pos_r, neg_h, neg_t, neg_r)

    def _finish(p_ref, o_ref):
        o_ref[0, 0] = jnp.sum(p_ref[...])

    loss = pl.pallas_call(
        _finish,
        out_shape=jax.ShapeDtypeStruct((1, 1), jnp.float32),
        in_specs=[pl.BlockSpec(memory_space=pltpu.VMEM)],
        out_specs=pl.BlockSpec(memory_space=pltpu.SMEM),
    )(partials)
    return loss[0, 0]


# speedup vs baseline: 1.8794x; 1.0453x over previous
import jax
import jax.numpy as jnp
from jax import lax
from jax.experimental import pallas as pl
from jax.experimental.pallas import tpu as pltpu
from jax.experimental.pallas import tpu_sc as plsc

_MARGIN = 2.0
_BATCH = 4096
_DIM = 128

_NC = 2
_NS = 16
_NW = _NC * _NS
_BPW = _BATCH // _NW
_LANES = 16
_NCHUNK = _DIM // _LANES
_UNROLL = 2

_TAKE_DNUMS = lax.GatherDimensionNumbers(
    offset_dims=(), collapsed_slice_dims=(0,), start_index_map=(0,))


def _take16(v, idx):
    return lax.gather(v, idx[:, None], _TAKE_DNUMS, slice_sizes=(1,),
                      mode=lax.GatherScatterMode.PROMISE_IN_BOUNDS)


def _sc_partials(ent_hbm, rel_hbm,
                 ph_hbm, pt_hbm, pr_hbm, nh_hbm, nt_hbm, nr_hbm,
                 out_hbm,
                 ph_v, pt_v, pr_v, nh_v, nt_v, nr_v,
                 h_v, t_v, r_v, hn_v, tn_v, rn_v,
                 dpos_v, res_v, sem_i, sem_p, sem_n):
    wid = lax.axis_index("s") * _NC + lax.axis_index("c")
    base = wid * _BPW
    sl = pl.ds(base, _BPW)

    i1 = pltpu.async_copy(ph_hbm.at[sl], ph_v, sem_i)
    i2 = pltpu.async_copy(pt_hbm.at[sl], pt_v, sem_i)
    i3 = pltpu.async_copy(pr_hbm.at[sl], pr_v, sem_i)
    i4 = pltpu.async_copy(nh_hbm.at[sl], nh_v, sem_i)
    i5 = pltpu.async_copy(nt_hbm.at[sl], nt_v, sem_i)
    i6 = pltpu.async_copy(nr_hbm.at[sl], nr_v, sem_i)
    i1.wait(); i2.wait(); i3.wait()
    c1 = pltpu.async_copy(ent_hbm.at[ph_v], h_v, sem_p)
    c2 = pltpu.async_copy(ent_hbm.at[pt_v], t_v, sem_p)
    c3 = pltpu.async_copy(rel_hbm.at[pr_v], r_v, sem_p)
    i4.wait(); i5.wait(); i6.wait()
    c4 = pltpu.async_copy(ent_hbm.at[nh_v], hn_v, sem_n)
    c5 = pltpu.async_copy(ent_hbm.at[nt_v], tn_v, sem_n)
    c6 = pltpu.async_copy(rel_hbm.at[nr_v], rn_v, sem_n)

    lane = lax.iota(jnp.int32, _LANES)
    rot = [(lane + s) % _LANES for s in (8, 4, 2, 1)]
    lane0 = lane == 0

    c1.wait(); c2.wait(); c3.wait()

    def pos_body(k, carry):
        for u in range(_UNROLL):
            i = k * _UNROLL + u
            acc = jnp.zeros((_LANES,), jnp.float32)
            for d in range(_NCHUNK):
                c = pl.ds(d * _LANES, _LANES)
                acc += jnp.abs(h_v[i, c] + r_v[i, c] - t_v[i, c])
            dpos_v[i, :] = acc
        return carry

    lax.fori_loop(0, _BPW // _UNROLL, pos_body, jnp.int32(0))

    c4.wait(); c5.wait(); c6.wait()

    def neg_body(k, loss_vec):
        for u in range(_UNROLL):
            i = k * _UNROLL + u
            acc = dpos_v[i, :]
            for d in range(_NCHUNK):
                c = pl.ds(d * _LANES, _LANES)
                acc -= jnp.abs(hn_v[i, c] + rn_v[i, c] - tn_v[i, c])
            for r in rot:
                acc = acc + _take16(acc, r)
            contrib = jnp.maximum(_MARGIN + acc, 0.0)
            loss_vec = loss_vec + jnp.where(lane0, contrib, 0.0)
        return loss_vec

    loss_vec = lax.fori_loop(0, _BPW // _UNROLL, neg_body,
                             jnp.zeros((_LANES,), jnp.float32))

    res_v[...] = loss_vec
    pltpu.sync_copy(res_v, out_hbm.at[wid])


@jax.jit
def kernel(entity_vec, relation_vec, pos_h, pos_t, pos_r, neg_h, neg_t, neg_r):
    mesh = plsc.VectorSubcoreMesh(core_axis_name="c", subcore_axis_name="s")
    partials = pl.kernel(
        _sc_partials,
        out_type=jax.ShapeDtypeStruct((_NW, _LANES), jnp.float32),
        mesh=mesh,
        scratch_types=[
            pltpu.VMEM((_BPW,), jnp.int32),
            pltpu.VMEM((_BPW,), jnp.int32),
            pltpu.VMEM((_BPW,), jnp.int32),
            pltpu.VMEM((_BPW,), jnp.int32),
            pltpu.VMEM((_BPW,), jnp.int32),
            pltpu.VMEM((_BPW,), jnp.int32),
            pltpu.VMEM((_BPW, _DIM), jnp.float32),
            pltpu.VMEM((_BPW, _DIM), jnp.float32),
            pltpu.VMEM((_BPW, _DIM), jnp.float32),
            pltpu.VMEM((_BPW, _DIM), jnp.float32),
            pltpu.VMEM((_BPW, _DIM), jnp.float32),
            pltpu.VMEM((_BPW, _DIM), jnp.float32),
            pltpu.VMEM((_BPW, _LANES), jnp.float32),
            pltpu.VMEM((_LANES,), jnp.float32),
            pltpu.SemaphoreType.DMA,
            pltpu.SemaphoreType.DMA,
            pltpu.SemaphoreType.DMA,
        ],
    )(entity_vec, relation_vec, pos_h, pos_t, pos_r, neg_h, neg_t, neg_r)

    def _finish(p_ref, o_ref):
        o_ref[0, 0] = jnp.sum(p_ref[...])

    loss = pl.pallas_call(
        _finish,
        out_shape=jax.ShapeDtypeStruct((1, 1), jnp.float32),
        in_specs=[pl.BlockSpec(memory_space=pltpu.VMEM)],
        out_specs=pl.BlockSpec(memory_space=pltpu.SMEM),
    )(partials)
    return loss[0, 0]


# speedup vs baseline: 1.8861x; 1.0035x over previous
import jax
import jax.numpy as jnp
from jax import lax
from jax.experimental import pallas as pl
from jax.experimental.pallas import tpu as pltpu
from jax.experimental.pallas import tpu_sc as plsc

_MARGIN = 2.0
_BATCH = 4096
_DIM = 128

_NC = 2
_NS = 16
_NW = _NC * _NS
_BPW = _BATCH // _NW
_LANES = 16
_NCHUNK = _DIM // _LANES
_UNROLL = 2

_TAKE_DNUMS = lax.GatherDimensionNumbers(
    offset_dims=(), collapsed_slice_dims=(0,), start_index_map=(0,))


def _take16(v, idx):
    return lax.gather(v, idx[:, None], _TAKE_DNUMS, slice_sizes=(1,),
                      mode=lax.GatherScatterMode.PROMISE_IN_BOUNDS)


def _sc_partials(ent_hbm, rel_hbm,
                 ph_hbm, pt_hbm, pr_hbm, nh_hbm, nt_hbm, nr_hbm,
                 out_hbm,
                 ph_v, pt_v, pr_v, nh_v, nt_v, nr_v,
                 h_v, t_v, r_v, hn_v, tn_v, rn_v,
                 dpos_v, res_v, sem_i, sem_p, sem_n):
    wid = lax.axis_index("s") * _NC + lax.axis_index("c")
    base = wid * _BPW
    sl = pl.ds(base, _BPW)

    i1 = pltpu.async_copy(ph_hbm.at[sl], ph_v, sem_i)
    i2 = pltpu.async_copy(pt_hbm.at[sl], pt_v, sem_i)
    i3 = pltpu.async_copy(pr_hbm.at[sl], pr_v, sem_i)
    i4 = pltpu.async_copy(nh_hbm.at[sl], nh_v, sem_i)
    i5 = pltpu.async_copy(nt_hbm.at[sl], nt_v, sem_i)
    i6 = pltpu.async_copy(nr_hbm.at[sl], nr_v, sem_i)
    i1.wait(); i2.wait(); i3.wait()
    c1 = pltpu.async_copy(ent_hbm.at[ph_v], h_v, sem_p)
    c2 = pltpu.async_copy(ent_hbm.at[pt_v], t_v, sem_p)
    c3 = pltpu.async_copy(rel_hbm.at[pr_v], r_v, sem_p)
    i4.wait(); i5.wait(); i6.wait()
    c4 = pltpu.async_copy(ent_hbm.at[nh_v], hn_v, sem_n)
    c5 = pltpu.async_copy(ent_hbm.at[nt_v], tn_v, sem_n)
    c6 = pltpu.async_copy(rel_hbm.at[nr_v], rn_v, sem_n)

    lane = lax.iota(jnp.int32, _LANES)
    rot8 = (lane + 8) % _LANES
    half_rots = [(lane & 8) | ((lane + s) & 7) for s in (4, 2, 1)]
    low_half = lane < 8
    lane08 = (lane & 7) == 0

    def _tree_l1(av, bv, cv, i):
        ch = []
        for d in range(_NCHUNK):
            c = pl.ds(d * _LANES, _LANES)
            ch.append(jnp.abs(av[i, c] + bv[i, c] - cv[i, c]))
        return ((ch[0] + ch[1]) + (ch[2] + ch[3])) + \
               ((ch[4] + ch[5]) + (ch[6] + ch[7]))

    c1.wait(); c2.wait(); c3.wait()

    def pos_body(k, carry):
        for u in range(_UNROLL):
            i = k * _UNROLL + u
            dpos_v[i, :] = _tree_l1(h_v, r_v, t_v, i)
        return carry

    lax.fori_loop(0, _BPW // _UNROLL, pos_body, jnp.int32(0))

    c4.wait(); c5.wait(); c6.wait()

    def neg_body(k, loss_vec):
        i = k * _UNROLL
        a = dpos_v[i, :] - _tree_l1(hn_v, rn_v, tn_v, i)
        b = dpos_v[i + 1, :] - _tree_l1(hn_v, rn_v, tn_v, i + 1)
        a2 = a + _take16(a, rot8)
        b2 = b + _take16(b, rot8)
        m = jnp.where(low_half, a2, b2)
        for r in half_rots:
            m = m + _take16(m, r)
        contrib = jnp.maximum(_MARGIN + m, 0.0)
        return loss_vec + jnp.where(lane08, contrib, 0.0)

    loss_vec = lax.fori_loop(0, _BPW // _UNROLL, neg_body,
                             jnp.zeros((_LANES,), jnp.float32))

    res_v[...] = loss_vec
    pltpu.sync_copy(res_v, out_hbm.at[wid])


@jax.jit
def kernel(entity_vec, relation_vec, pos_h, pos_t, pos_r, neg_h, neg_t, neg_r):
    mesh = plsc.VectorSubcoreMesh(core_axis_name="c", subcore_axis_name="s")
    partials = pl.kernel(
        _sc_partials,
        out_type=jax.ShapeDtypeStruct((_NW, _LANES), jnp.float32),
        mesh=mesh,
        scratch_types=[
            pltpu.VMEM((_BPW,), jnp.int32),
            pltpu.VMEM((_BPW,), jnp.int32),
            pltpu.VMEM((_BPW,), jnp.int32),
            pltpu.VMEM((_BPW,), jnp.int32),
            pltpu.VMEM((_BPW,), jnp.int32),
            pltpu.VMEM((_BPW,), jnp.int32),
            pltpu.VMEM((_BPW, _DIM), jnp.float32),
            pltpu.VMEM((_BPW, _DIM), jnp.float32),
            pltpu.VMEM((_BPW, _DIM), jnp.float32),
            pltpu.VMEM((_BPW, _DIM), jnp.float32),
            pltpu.VMEM((_BPW, _DIM), jnp.float32),
            pltpu.VMEM((_BPW, _DIM), jnp.float32),
            pltpu.VMEM((_BPW, _LANES), jnp.float32),
            pltpu.VMEM((_LANES,), jnp.float32),
            pltpu.SemaphoreType.DMA,
            pltpu.SemaphoreType.DMA,
            pltpu.SemaphoreType.DMA,
        ],
    )(entity_vec, relation_vec, pos_h, pos_t, pos_r, neg_h, neg_t, neg_r)

    def _finish(p_ref, o_ref):
        o_ref[0, 0] = jnp.sum(p_ref[...])

    loss = pl.pallas_call(
        _finish,
        out_shape=jax.ShapeDtypeStruct((1, 1), jnp.float32),
        in_specs=[pl.BlockSpec(memory_space=pltpu.VMEM)],
        out_specs=pl.BlockSpec(memory_space=pltpu.SMEM),
    )(partials)
    return loss[0, 0]
